# baseline (speedup 1.0000x reference)
import jax
import jax.numpy as jnp
from jax import lax
from jax.experimental import pallas as pl
from jax.experimental.pallas import tpu as pltpu
from jax.experimental.pallas import tpu_sc as plsc

_TILE_N = 5120


def _sc_gather(attr_weights, attributes):
    b = attributes.shape[0]
    a, r = attr_weights.shape
    try:
        info = plsc.get_sparse_core_info()
        nc, ns = info.num_cores, info.num_subcores
    except Exception:
        nc, ns = 2, 16
    nw = nc * ns
    b_per_w = b // nw
    mesh = plsc.VectorSubcoreMesh(core_axis_name="c", subcore_axis_name="s")

    def body(table_hbm, idx_hbm, out_hbm, idx_v, rows_v, sem):
        wid = lax.axis_index("s") * nc + lax.axis_index("c")
        base = wid * b_per_w
        pltpu.sync_copy(idx_hbm.at[pl.ds(base, b_per_w)], idx_v)
        pltpu.async_copy(table_hbm.at[idx_v], rows_v, sem).wait()
        pltpu.sync_copy(rows_v, out_hbm.at[pl.ds(base, b_per_w)])

    k = pl.kernel(
        body,
        out_type=jax.ShapeDtypeStruct((b, r), jnp.float32),
        mesh=mesh,
        compiler_params=pltpu.CompilerParams(use_tc_tiling_on_sc=False),
        scratch_types=[
            pltpu.VMEM((b_per_w,), jnp.int32),
            pltpu.VMEM((b_per_w, r), jnp.float32),
            pltpu.SemaphoreType.DMA,
        ],
    )
    return k(attr_weights, attributes)


def _mm_body(g_ref, ent_ref, out_ref):
    out_ref[...] = jax.lax.dot_general(
        g_ref[...], ent_ref[...], (((1,), (1,)), ((), ())),
        preferred_element_type=jnp.float32)


def kernel(ent_emb, attr_weights, attributes):
    n, r = ent_emb.shape
    b = attributes.shape[0]
    gathered = _sc_gather(attr_weights, attributes)
    return pl.pallas_call(
        _mm_body,
        grid=(pl.cdiv(n, _TILE_N),),
        in_specs=[
            pl.BlockSpec((b, r), lambda i: (0, 0)),
            pl.BlockSpec((_TILE_N, r), lambda i: (i, 0)),
        ],
        out_specs=pl.BlockSpec((b, _TILE_N), lambda i: (0, i)),
        out_shape=jax.ShapeDtypeStruct((b, n), jnp.float32),
    )(gathered, ent_emb)
